# Initial kernel scaffold; baseline (speedup 1.0000x reference)
import jax
import jax.numpy as jnp
from jax.experimental import pallas as pl


def kernel(x, edge_index, xl, hyper_adj, Wl, bl, aA1, ab1, aA2, Wc1, bc1, c1_Wq, c1_bq, c1_Wk, c1_bk, c1_Wv, c1_bv, c1_Wd, c1_bd, c1_lnw, c1_lnb, Wc2, bc2, c2_Wq, c2_bq, c2_Wk, c2_bk, c2_Wv, c2_bv, c2_Wd, c2_bd, c2_lnw, c2_lnb, T_R):
    raise NotImplementedError("write your pallas kernel here")


# speedup vs baseline: 3.7894x; 3.7894x over previous
import functools

import jax
import jax.numpy as jnp
from jax import lax
from jax.experimental import pallas as pl
from jax.experimental.pallas import tpu as pltpu
from jax.experimental.pallas import tpu_sc as plsc

N = 4096
E = 131072
IN_FEAT = 256
N_HID = 128
N_CLASS = 64
DIM_LLMS = 1024

RB = 256
GRID = N // RB
SC_CH = 128
NW = 32
EPW = E // NW


def _sc_mesh():
    return plsc.VectorSubcoreMesh(core_axis_name="c", subcore_axis_name="s")


def _sc_degrees(src, dst, eio_rows, z128):
    @functools.partial(
        pl.kernel,
        out_type=jax.ShapeDtypeStruct((2 * N, N_HID), jnp.float32),
        mesh=_sc_mesh(),
        scratch_types=[
            pltpu.VMEM((SC_CH,), jnp.int32),
            pltpu.VMEM((SC_CH,), jnp.int32),
            pltpu.VMEM((2 * SC_CH, N_HID), jnp.float32),
            pltpu.VMEM_SHARED((N, N_HID), jnp.float32),
            pltpu.SemaphoreType.DMA,
        ],
    )
    def deg_kernel(src_hbm, dst_hbm, eio_hbm, z_hbm, deg_hbm,
                   sidx, didx, eio_v, acc, sem):
        cid = lax.axis_index("c")
        sid = lax.axis_index("s")
        wid = sid * 2 + cid
        rows = N // 16
        pltpu.sync_copy(z_hbm.at[pl.ds(sid * rows, rows)],
                        acc.at[pl.ds(sid * rows, rows)])
        pltpu.sync_copy(eio_hbm, eio_v)
        plsc.subcore_barrier()
        base = wid * EPW

        @pl.loop(0, EPW // SC_CH)
        def _(k):
            off = base + k * SC_CH
            pltpu.sync_copy(src_hbm.at[pl.ds(off, SC_CH)], sidx)
            pltpu.sync_copy(dst_hbm.at[pl.ds(off, SC_CH)], didx)
            pltpu.sync_copy(eio_v.at[pl.ds(0, SC_CH)], acc.at[sidx], add=True)
            pltpu.sync_copy(eio_v.at[pl.ds(SC_CH, SC_CH)], acc.at[didx],
                            add=True)

        plsc.subcore_barrier()
        pltpu.sync_copy(acc.at[pl.ds(sid * rows, rows)],
                        deg_hbm.at[pl.ds(cid * N + sid * rows, rows)])

    return deg_kernel(src, dst, eio_rows, z128)


def _sc_segment_sum(mrows, src, dst, zD, D=N_HID):
    @functools.partial(
        pl.kernel,
        out_type=jax.ShapeDtypeStruct((2 * N, D), jnp.float32),
        mesh=_sc_mesh(),
        scratch_types=[
            pltpu.VMEM((SC_CH,), jnp.int32),
            pltpu.VMEM((SC_CH,), jnp.int32),
            pltpu.VMEM((SC_CH, D), jnp.float32),
            pltpu.VMEM_SHARED((N, D), jnp.float32),
            pltpu.SemaphoreType.DMA,
        ],
    )
    def seg_kernel(m_hbm, src_hbm, dst_hbm, z_hbm, out_hbm,
                   sidx, didx, rows_v, acc, sem):
        cid = lax.axis_index("c")
        sid = lax.axis_index("s")
        wid = sid * 2 + cid
        rows = N // 16
        pltpu.sync_copy(z_hbm.at[pl.ds(sid * rows, rows)],
                        acc.at[pl.ds(sid * rows, rows)])
        plsc.subcore_barrier()
        base = wid * EPW

        @pl.loop(0, EPW // SC_CH)
        def _(k):
            off = base + k * SC_CH
            pltpu.sync_copy(src_hbm.at[pl.ds(off, SC_CH)], sidx)
            pltpu.sync_copy(dst_hbm.at[pl.ds(off, SC_CH)], didx)
            pltpu.async_copy(m_hbm.at[sidx], rows_v, sem).wait()
            pltpu.sync_copy(rows_v, acc.at[didx], add=True)

        plsc.subcore_barrier()
        pltpu.sync_copy(acc.at[pl.ds(sid * rows, rows)],
                        out_hbm.at[pl.ds(cid * N + sid * rows, rows)])

    return seg_kernel(mrows, src, dst, zD)


def _fusion_body(xl_ref, x_ref, Wl_ref, bl_ref, aA1_ref, ab1_ref, aA2_ref,
                 Wc1_ref, deg_ref, out_ref):
    i = pl.program_id(0)
    xb = x_ref[...]
    n_t = jnp.dot(xl_ref[...], Wl_ref[...],
                  preferred_element_type=jnp.float32) + bl_ref[...]
    aA1 = aA1_ref[...]
    ab1 = ab1_ref[...]
    aA2 = aA2_ref[...]
    w1 = jnp.sum(jnp.maximum(jnp.dot(xb, aA1,
                                     preferred_element_type=jnp.float32)
                             + ab1, 0.0) * aA2, axis=1, keepdims=True)
    w2 = jnp.sum(jnp.maximum(jnp.dot(n_t, aA1,
                                     preferred_element_type=jnp.float32)
                             + ab1, 0.0) * aA2, axis=1, keepdims=True)
    m = jnp.maximum(w1, w2)
    e1 = jnp.exp(w1 - m)
    e2 = jnp.exp(w2 - m)
    h0 = (e1 * xb + e2 * n_t) / (e1 + e2)
    dsum = (deg_ref[pl.ds(i * RB, RB), 0:1]
            + deg_ref[pl.ds(N + i * RB, RB), 0:1])
    s_out = lax.rsqrt(jnp.maximum(dsum, 1.0))
    out_ref[...] = jnp.dot(h0, Wc1_ref[...],
                           preferred_element_type=jnp.float32) * s_out


def _fusion_call(x, xl, Wl, bl, aA1, ab1, aA2, Wc1, deg_p):
    return pl.pallas_call(
        _fusion_body,
        grid=(GRID,),
        in_specs=[
            pl.BlockSpec((RB, DIM_LLMS), lambda i: (i, 0)),
            pl.BlockSpec((RB, IN_FEAT), lambda i: (i, 0)),
            pl.BlockSpec((DIM_LLMS, IN_FEAT), lambda i: (0, 0)),
            pl.BlockSpec((1, IN_FEAT), lambda i: (0, 0)),
            pl.BlockSpec((IN_FEAT, 64), lambda i: (0, 0)),
            pl.BlockSpec((1, 64), lambda i: (0, 0)),
            pl.BlockSpec((1, 64), lambda i: (0, 0)),
            pl.BlockSpec((IN_FEAT, N_HID), lambda i: (0, 0)),
            pl.BlockSpec((2 * N, N_HID), lambda i: (0, 0)),
        ],
        out_specs=pl.BlockSpec((RB, N_HID), lambda i: (i, 0)),
        out_shape=jax.ShapeDtypeStruct((N, N_HID), jnp.float32),
    )(xl, x, Wl, bl, aA1, ab1, aA2, Wc1, deg_p)


def _crd_body(parts_ref, deg_ref, bc1_ref,
              Wq_ref, bq_ref, Wk_ref, bk_ref, Wv_ref, bv_ref,
              Wd_ref, bd_ref, lnw_ref, lnb_ref, Wc2_ref,
              out_ref, X_scr, KT_scr, V_scr):
    i = pl.program_id(0)

    @pl.when(i == 0)
    def _():
        din = deg_ref[0:N, 1:2] + deg_ref[N:2 * N, 1:2]
        s_in = lax.rsqrt(jnp.maximum(din, 1.0))
        X = (parts_ref[0:N, :] + parts_ref[N:2 * N, :]) * s_in + bc1_ref[...]
        X_scr[...] = X
        K = jnp.dot(X, Wk_ref[...],
                    preferred_element_type=jnp.float32) + bk_ref[...]
        KT_scr[...] = K.T
        V_scr[...] = jnp.dot(X, Wv_ref[...],
                             preferred_element_type=jnp.float32) + bv_ref[...]

    xb = X_scr[pl.ds(i * RB, RB), :]
    q = jnp.dot(xb, Wq_ref[...],
                preferred_element_type=jnp.float32) + bq_ref[...]
    KT = KT_scr[...]
    V = V_scr[...]
    col = lax.broadcasted_iota(jnp.int32, (1, N_HID), 1) // 32
    scale = 1.0 / (32.0 ** 0.5)
    ctx = jnp.zeros((RB, N_HID), jnp.float32)
    for h in range(4):
        mask = (col == h)
        qm = jnp.where(mask, q, 0.0)
        s = jnp.dot(qm, KT, preferred_element_type=jnp.float32) * scale
        mrow = jnp.max(s, axis=1, keepdims=True)
        p = jnp.exp(s - mrow)
        p = p / jnp.sum(p, axis=1, keepdims=True)
        pv = jnp.dot(p, V, preferred_element_type=jnp.float32)
        ctx = ctx + jnp.where(mask, pv, 0.0)
    o = jnp.dot(ctx, Wd_ref[...],
                preferred_element_type=jnp.float32) + bd_ref[...] + xb
    mu = jnp.mean(o, axis=1, keepdims=True)
    var = jnp.mean((o - mu) ** 2, axis=1, keepdims=True)
    o = lnw_ref[...] * ((o - mu) * lax.rsqrt(var + 1e-12)) + lnb_ref[...]
    o = jnp.maximum(o, 0.0)
    dsum = (deg_ref[pl.ds(i * RB, RB), 0:1]
            + deg_ref[pl.ds(N + i * RB, RB), 0:1])
    s_out = lax.rsqrt(jnp.maximum(dsum, 1.0))
    m2 = jnp.dot(o, Wc2_ref[...],
                 preferred_element_type=jnp.float32) * s_out
    out_ref[...] = jnp.concatenate(
        [m2, jnp.zeros((RB, N_HID - N_CLASS), jnp.float32)], axis=1)


def _crd_call(parts, deg_p, bc1, Wq, bq, Wk, bk, Wv, bv,
              Wd, bd, lnw, lnb, Wc2):
    full = lambda shape: pl.BlockSpec(shape, lambda i: tuple(0 for _ in shape))
    return pl.pallas_call(
        _crd_body,
        grid=(GRID,),
        in_specs=[
            full((2 * N, N_HID)),
            full((2 * N, N_HID)),
            full((1, N_HID)),
            full((N_HID, N_HID)), full((1, N_HID)),
            full((N_HID, N_HID)), full((1, N_HID)),
            full((N_HID, N_HID)), full((1, N_HID)),
            full((N_HID, N_HID)), full((1, N_HID)),
            full((1, N_HID)), full((1, N_HID)),
            full((N_HID, N_CLASS)),
        ],
        out_specs=pl.BlockSpec((RB, N_HID), lambda i: (i, 0)),
        out_shape=jax.ShapeDtypeStruct((N, N_HID), jnp.float32),
        scratch_shapes=[
            pltpu.VMEM((N, N_HID), jnp.float32),
            pltpu.VMEM((N_HID, N), jnp.float32),
            pltpu.VMEM((N, N_HID), jnp.float32),
        ],
    )(parts, deg_p, bc1, Wq, bq, Wk, bk, Wv, bv, Wd, bd,
      lnw, lnb, Wc2)


def _cls_body(parts_ref, deg_ref, bc2_ref,
              Wq_ref, bq_ref, Wk_ref, bk_ref, Wv_ref, bv_ref,
              Wd_ref, bd_ref, lnw_ref, lnb_ref,
              out_ref, X_scr, KT_scr, V_scr):
    i = pl.program_id(0)

    @pl.when(i == 0)
    def _():
        din = deg_ref[0:N, 1:2] + deg_ref[N:2 * N, 1:2]
        s_in = lax.rsqrt(jnp.maximum(din, 1.0))
        X = (parts_ref[0:N, 0:N_CLASS]
             + parts_ref[N:2 * N, 0:N_CLASS]) * s_in + bc2_ref[...]
        X_scr[...] = X
        K = jnp.dot(X, Wk_ref[...],
                    preferred_element_type=jnp.float32) + bk_ref[...]
        KT_scr[...] = K.T
        V_scr[...] = jnp.dot(X, Wv_ref[...],
                             preferred_element_type=jnp.float32) + bv_ref[...]

    xb = X_scr[pl.ds(i * RB, RB), :]
    q = jnp.dot(xb, Wq_ref[...],
                preferred_element_type=jnp.float32) + bq_ref[...]
    s = jnp.dot(q, KT_scr[...],
                preferred_element_type=jnp.float32) * (1.0 / 8.0)
    mrow = jnp.max(s, axis=1, keepdims=True)
    p = jnp.exp(s - mrow)
    p = p / jnp.sum(p, axis=1, keepdims=True)
    ctx = jnp.dot(p, V_scr[...], preferred_element_type=jnp.float32)
    o = jnp.dot(ctx, Wd_ref[...],
                preferred_element_type=jnp.float32) + bd_ref[...] + xb
    mu = jnp.mean(o, axis=1, keepdims=True)
    var = jnp.mean((o - mu) ** 2, axis=1, keepdims=True)
    o = lnw_ref[...] * ((o - mu) * lax.rsqrt(var + 1e-12)) + lnb_ref[...]
    mo = jnp.max(o, axis=1, keepdims=True)
    eo = jnp.exp(o - mo)
    out_ref[...] = eo / jnp.sum(eo, axis=1, keepdims=True)


def _cls_call(parts, deg_p, bc2, Wq, bq, Wk, bk, Wv, bv, Wd, bd, lnw, lnb):
    full = lambda shape: pl.BlockSpec(shape, lambda i: tuple(0 for _ in shape))
    return pl.pallas_call(
        _cls_body,
        grid=(GRID,),
        in_specs=[
            full((2 * N, N_HID)),
            full((2 * N, N_HID)),
            full((1, N_CLASS)),
            full((N_CLASS, N_CLASS)), full((1, N_CLASS)),
            full((N_CLASS, N_CLASS)), full((1, N_CLASS)),
            full((N_CLASS, N_CLASS)), full((1, N_CLASS)),
            full((N_CLASS, N_CLASS)), full((1, N_CLASS)),
            full((1, N_CLASS)), full((1, N_CLASS)),
        ],
        out_specs=pl.BlockSpec((RB, N_CLASS), lambda i: (i, 0)),
        out_shape=jax.ShapeDtypeStruct((N, N_CLASS), jnp.float32),
        scratch_shapes=[
            pltpu.VMEM((N, N_CLASS), jnp.float32),
            pltpu.VMEM((N_CLASS, N), jnp.float32),
            pltpu.VMEM((N, N_CLASS), jnp.float32),
        ],
    )(parts, deg_p, bc2, Wq, bq, Wk, bk, Wv, bv, Wd, bd, lnw, lnb)


def _hyper_body(h_ref, xo_ref, TR_ref, out_ref):
    t = jnp.dot(h_ref[...], xo_ref[...], preferred_element_type=jnp.float32)
    t = jnp.dot(t, TR_ref[...], preferred_element_type=jnp.float32)
    m = jnp.max(t, axis=1, keepdims=True)
    e = jnp.exp(t - m)
    out_ref[...] = e / jnp.sum(e, axis=1, keepdims=True)


def _hyper_call(hyper_adj, x_out, T_R):
    return pl.pallas_call(
        _hyper_body,
        grid=(GRID,),
        in_specs=[
            pl.BlockSpec((RB, N), lambda i: (i, 0)),
            pl.BlockSpec((N, N_CLASS), lambda i: (0, 0)),
            pl.BlockSpec((N_CLASS, IN_FEAT), lambda i: (0, 0)),
        ],
        out_specs=pl.BlockSpec((RB, IN_FEAT), lambda i: (i, 0)),
        out_shape=jax.ShapeDtypeStruct((N, IN_FEAT), jnp.float32),
    )(hyper_adj, x_out, T_R)


def kernel(x, edge_index, xl, hyper_adj, Wl, bl, aA1, ab1, aA2, Wc1, bc1,
           c1_Wq, c1_bq, c1_Wk, c1_bk, c1_Wv, c1_bv, c1_Wd, c1_bd,
           c1_lnw, c1_lnb, Wc2, bc2,
           c2_Wq, c2_bq, c2_Wk, c2_bk, c2_Wv, c2_bv, c2_Wd, c2_bd,
           c2_lnw, c2_lnb, T_R):
    src = edge_index[0]
    dst = edge_index[1]
    z128 = jnp.zeros((N, N_HID), jnp.float32)
    col = jnp.arange(N_HID)
    eio_rows = jnp.concatenate(
        [jnp.broadcast_to((col == 0).astype(jnp.float32), (SC_CH, N_HID)),
         jnp.broadcast_to((col == 1).astype(jnp.float32), (SC_CH, N_HID))],
        axis=0)
    r = lambda v: v.reshape(1, -1)

    deg_p = _sc_degrees(src, dst, eio_rows, z128)
    m1 = _fusion_call(x, xl, Wl, r(bl), aA1, r(ab1), r(aA2), Wc1, deg_p)
    agg1 = _sc_segment_sum(m1, src, dst, z128)
    m2 = _crd_call(agg1, deg_p, r(bc1),
                   c1_Wq, r(c1_bq), c1_Wk, r(c1_bk), c1_Wv, r(c1_bv),
                   c1_Wd, r(c1_bd), r(c1_lnw), r(c1_lnb), Wc2)
    agg2 = _sc_segment_sum(m2, src, dst, z128)
    x_out = _cls_call(agg2, deg_p, r(bc2),
                      c2_Wq, r(c2_bq), c2_Wk, r(c2_bk), c2_Wv, r(c2_bv),
                      c2_Wd, r(c2_bd), r(c2_lnw), r(c2_lnb))
    hyper_out = _hyper_call(hyper_adj, x_out, T_R)
    return (x_out, hyper_out)
